# Initial kernel scaffold; baseline (speedup 1.0000x reference)
import jax
import jax.numpy as jnp
from jax.experimental import pallas as pl


def kernel(x, edge_index, edge_attr, W1, b1, W2, b2):
    raise NotImplementedError("write your pallas kernel here")


# speedup vs baseline: 2.9300x; 2.9300x over previous
import functools

import jax
import jax.numpy as jnp
from jax import lax
from jax.experimental import pallas as pl
from jax.experimental.pallas import tpu as pltpu
from jax.experimental.pallas import tpu_sc as plsc

NC, NS = 2, 16
NW = NC * NS
GCH = 80

N = 10000
E = 320000
D = 128
DE = 16
H = 128
DI = 16

EPW = E // NW
RPW = EPW // GCH
ZR = 1000
NZW = N // ZR


def _xa_body(x_ref, w_ref, b_ref, o_ref):
    o_ref[...] = (
        jnp.dot(x_ref[...], w_ref[...], preferred_element_type=jnp.float32)
        + b_ref[...]
    )


def _mlp_body(g_ref, ea_ref, w1b_ref, w2_ref, b2_ref, o_ref):
    h = jnp.maximum(
        g_ref[...]
        + jnp.dot(ea_ref[...], w1b_ref[...], preferred_element_type=jnp.float32),
        0.0,
    )
    o_ref[...] = jnp.dot(h, w2_ref[...], preferred_element_type=jnp.float32) + b2_ref[...]


def _div_body(s_ref, c_ref, o_ref):
    sm = s_ref[0] + s_ref[1]
    ct = c_ref[0] + c_ref[1]
    o_ref[...] = sm / jnp.maximum(ct, 1.0)


def _gather_body(xa_hbm, idx_hbm, g_hbm, idx_v, rows_v, sem):
    c = lax.axis_index("c")
    s = lax.axis_index("s")
    base = (c * NS + s) * EPW

    def step(i, carry):
        e0 = base + i * GCH
        pltpu.sync_copy(idx_hbm.at[pl.ds(e0, GCH)], idx_v)
        pltpu.async_copy(xa_hbm.at[idx_v], rows_v, sem).wait()
        pltpu.sync_copy(rows_v, g_hbm.at[pl.ds(e0, GCH)])
        return carry

    lax.fori_loop(0, RPW, step, 0)


def _scatter_body(col_hbm, en_hbm, sums_hbm, cnts_hbm,
                  idx_v, dat_v, one_v, zer_v, acc_s, acc_c):
    c = lax.axis_index("c")
    s = lax.axis_index("s")
    base = (c * NS + s) * EPW

    def fill_z(i, carry):
        zer_v[i] = jnp.zeros((16,), jnp.float32)
        return carry

    lax.fori_loop(0, ZR, fill_z, 0)

    def fill_o(i, carry):
        one_v[i] = jnp.full((16,), 1.0, jnp.float32)
        return carry

    lax.fori_loop(0, GCH, fill_o, 0)

    @pl.when(s < NZW)
    def _():
        pltpu.sync_copy(zer_v, acc_s.at[pl.ds(s * ZR, ZR)])
        pltpu.sync_copy(zer_v, acc_c.at[pl.ds(s * ZR, ZR)])

    plsc.subcore_barrier()

    def step(i, carry):
        e0 = base + i * GCH
        pltpu.sync_copy(col_hbm.at[pl.ds(e0, GCH)], idx_v)
        pltpu.sync_copy(en_hbm.at[pl.ds(e0, GCH)], dat_v)
        pltpu.sync_copy(dat_v, acc_s.at[idx_v], add=True)
        pltpu.sync_copy(one_v, acc_c.at[idx_v], add=True)
        return carry

    lax.fori_loop(0, RPW, step, 0)
    plsc.subcore_barrier()

    @pl.when(s < NZW)
    def _():
        pltpu.sync_copy(acc_s.at[pl.ds(s * ZR, ZR)], sums_hbm.at[c, pl.ds(s * ZR, ZR)])
        pltpu.sync_copy(acc_c.at[pl.ds(s * ZR, ZR)], cnts_hbm.at[c, pl.ds(s * ZR, ZR)])


_sc_mesh = plsc.VectorSubcoreMesh(
    core_axis_name="c", subcore_axis_name="s", num_cores=NC, num_subcores=NS
)

_sc_params = pltpu.CompilerParams(use_tc_tiling_on_sc=False)

_gather = pl.kernel(
    _gather_body,
    compiler_params=_sc_params,
    out_type=jax.ShapeDtypeStruct((E, H), jnp.float32),
    mesh=_sc_mesh,
    scratch_types=[
        pltpu.VMEM((GCH,), jnp.int32),
        pltpu.VMEM((GCH, H), jnp.float32),
        pltpu.SemaphoreType.DMA,
    ],
)

_scatter = pl.kernel(
    _scatter_body,
    compiler_params=_sc_params,
    out_type=(
        jax.ShapeDtypeStruct((NC, N, DI), jnp.float32),
        jax.ShapeDtypeStruct((NC, N, DI), jnp.float32),
    ),
    mesh=_sc_mesh,
    scratch_types=[
        pltpu.VMEM((GCH,), jnp.int32),
        pltpu.VMEM((GCH, DI), jnp.float32),
        pltpu.VMEM((GCH, DI), jnp.float32),
        pltpu.VMEM((ZR, DI), jnp.float32),
        pltpu.VMEM_SHARED((N, DI), jnp.float32),
        pltpu.VMEM_SHARED((N, DI), jnp.float32),
    ],
)

BE = 4000

_mlp = pl.pallas_call(
    _mlp_body,
    grid=(E // BE,),
    in_specs=[
        pl.BlockSpec((BE, H), lambda i: (i, 0)),
        pl.BlockSpec((BE, DE), lambda i: (i, 0)),
        pl.BlockSpec((DE, H), lambda i: (0, 0)),
        pl.BlockSpec((H, DI), lambda i: (0, 0)),
        pl.BlockSpec((1, DI), lambda i: (0, 0)),
    ],
    out_specs=pl.BlockSpec((BE, DI), lambda i: (i, 0)),
    out_shape=jax.ShapeDtypeStruct((E, DI), jnp.float32),
)

_xa = pl.pallas_call(
    _xa_body,
    out_shape=jax.ShapeDtypeStruct((N, H), jnp.float32),
)

_div = pl.pallas_call(
    _div_body,
    out_shape=jax.ShapeDtypeStruct((N, DI), jnp.float32),
)


def kernel(x, edge_index, edge_attr, W1, b1, W2, b2):
    row = edge_index[0]
    col = edge_index[1]
    xa = _xa(x, W1[:D], b1.reshape(1, H))
    g = _gather(xa, row)
    edge_new = _mlp(g, edge_attr, W1[D:], W2, b2.reshape(1, DI))
    sums, cnts = _scatter(col, edge_new)
    node_new = _div(sums, cnts)
    return (node_new, edge_new)


# speedup vs baseline: 3.5276x; 1.2039x over previous
import jax
import jax.numpy as jnp
from jax import lax
from jax.experimental import pallas as pl
from jax.experimental.pallas import tpu as pltpu
from jax.experimental.pallas import tpu_sc as plsc

NC, NS = 2, 16
NW = NC * NS
STR = 80
NSTR = 5
CH = STR * NSTR
NCHUNK = 25

N = 10000
E = 320000
D = 128
DE = 16
H = 128
DI = 16

EPW = E // NW
ZR = 1000
NZW = N // ZR

EW = E // 8
BW = 400


def _xa_body(x_ref, w_ref, b_ref, o_ref):
    o_ref[...] = (
        jnp.dot(x_ref[...], w_ref[...], preferred_element_type=jnp.float32)
        + b_ref[...]
    )


def _mlp_body(g_ref, ea_ref, w1big_ref, w2big_ref, b2w_ref, o_ref):
    eb = jnp.dot(ea_ref[...], w1big_ref[...], preferred_element_type=jnp.float32)
    h = jnp.maximum(g_ref[...] + eb, 0.0)
    o_ref[...] = (
        jnp.dot(h, w2big_ref[...], preferred_element_type=jnp.float32)
        + b2w_ref[...]
    )


def _div_body(s_ref, c_ref, o_ref):
    sm = s_ref[0] + s_ref[1]
    ct = c_ref[0] + c_ref[1]
    o_ref[...] = sm / jnp.maximum(ct, 1.0)


def _gather_body(xa_hbm, idx_hbm, g_hbm, idx_v, rows_v, gsem, wsem):
    c = lax.axis_index("c")
    s = lax.axis_index("s")
    base = (c * NS + s) * EPW

    wb = [None, None]
    for t in range(NCHUNK):
        b = t % 2
        if t >= 2:
            wb[b].wait()
        e0 = base + t * CH
        pltpu.sync_copy(idx_hbm.at[pl.ds(e0, CH)], idx_v.at[b])
        fired = []
        for k in range(NSTR):
            fired.append(pltpu.async_copy(
                xa_hbm.at[idx_v.at[b].at[pl.ds(k * STR, STR)]],
                rows_v.at[b].at[pl.ds(k * STR, STR)],
                gsem.at[b],
            ))
        for f in fired:
            f.wait()
        wb[b] = pltpu.async_copy(rows_v.at[b], g_hbm.at[pl.ds(e0, CH)], wsem.at[b])
    wb[(NCHUNK - 2) % 2].wait()
    wb[(NCHUNK - 1) % 2].wait()


def _scatter_body(col_hbm, en_hbm, sums_hbm, cnts_hbm,
                  cidx_v, dat_v, one_v, zer_v, acc_s, acc_c, ssem, asem):
    c = lax.axis_index("c")
    s = lax.axis_index("s")
    base = (c * NS + s) * EPW

    def fill_z(i, carry):
        zer_v[i] = jnp.zeros((16,), jnp.float32)
        return carry

    lax.fori_loop(0, ZR, fill_z, 0)

    def fill_o(i, carry):
        one_v[i] = jnp.full((16,), 1.0, jnp.float32)
        return carry

    lax.fori_loop(0, STR, fill_o, 0)

    @pl.when(s < NZW)
    def _():
        pltpu.sync_copy(zer_v, acc_s.at[pl.ds(s * ZR, ZR)])
        pltpu.sync_copy(zer_v, acc_c.at[pl.ds(s * ZR, ZR)])

    plsc.subcore_barrier()

    adds = [None, None]
    for t in range(NCHUNK):
        b = t % 2
        if t >= 2:
            for f in adds[b]:
                f.wait()
        e0 = base + t * CH
        chunk = (c * NS + s) * NCHUNK + t
        st1 = pltpu.async_copy(col_hbm.at[chunk], cidx_v.at[b], ssem.at[b])
        st2 = pltpu.async_copy(en_hbm.at[pl.ds(e0, CH)], dat_v.at[b], ssem.at[b])
        st1.wait()
        st2.wait()
        fired = []
        for k in range(NSTR):
            ck = cidx_v.at[b].at[k]
            fired.append(pltpu.async_copy(
                dat_v.at[b].at[pl.ds(k * STR, STR)],
                acc_s.at[ck], asem.at[b], add=True))
            fired.append(pltpu.async_copy(
                one_v, acc_c.at[ck], asem.at[b], add=True))
        adds[b] = fired
    for b in ((NCHUNK - 2) % 2, (NCHUNK - 1) % 2):
        for f in adds[b]:
            f.wait()
    plsc.subcore_barrier()

    @pl.when(s < NZW)
    def _():
        pltpu.sync_copy(acc_s.at[pl.ds(s * ZR, ZR)], sums_hbm.at[c, pl.ds(s * ZR, ZR)])
        pltpu.sync_copy(acc_c.at[pl.ds(s * ZR, ZR)], cnts_hbm.at[c, pl.ds(s * ZR, ZR)])


_sc_mesh = plsc.VectorSubcoreMesh(
    core_axis_name="c", subcore_axis_name="s", num_cores=NC, num_subcores=NS
)
_sc_params = pltpu.CompilerParams(use_tc_tiling_on_sc=False)

_gather = pl.kernel(
    _gather_body,
    compiler_params=_sc_params,
    out_type=jax.ShapeDtypeStruct((E, H), jnp.float32),
    mesh=_sc_mesh,
    scratch_types=[
        pltpu.VMEM((2, CH), jnp.int32),
        pltpu.VMEM((2, CH, H), jnp.float32),
        pltpu.SemaphoreType.DMA((2,)),
        pltpu.SemaphoreType.DMA((2,)),
    ],
)

_scatter = pl.kernel(
    _scatter_body,
    compiler_params=_sc_params,
    out_type=(
        jax.ShapeDtypeStruct((NC, N, DI), jnp.float32),
        jax.ShapeDtypeStruct((NC, N, DI), jnp.float32),
    ),
    mesh=_sc_mesh,
    scratch_types=[
        pltpu.VMEM((2, NSTR, STR), jnp.int32),
        pltpu.VMEM((2, CH, DI), jnp.float32),
        pltpu.VMEM((STR, DI), jnp.float32),
        pltpu.VMEM((ZR, DI), jnp.float32),
        pltpu.VMEM_SHARED((N, DI), jnp.float32),
        pltpu.VMEM_SHARED((N, DI), jnp.float32),
        pltpu.SemaphoreType.DMA((2,)),
        pltpu.SemaphoreType.DMA((2,)),
    ],
)

_mlp = pl.pallas_call(
    _mlp_body,
    grid=(EW // BW,),
    in_specs=[
        pl.BlockSpec((BW, 8 * H), lambda i: (i, 0)),
        pl.BlockSpec((BW, 8 * DE), lambda i: (i, 0)),
        pl.BlockSpec((8 * DE, 8 * H), lambda i: (0, 0)),
        pl.BlockSpec((8 * H, 8 * DI), lambda i: (0, 0)),
        pl.BlockSpec((1, 8 * DI), lambda i: (0, 0)),
    ],
    out_specs=pl.BlockSpec((BW, 8 * DI), lambda i: (i, 0)),
    out_shape=jax.ShapeDtypeStruct((EW, 8 * DI), jnp.float32),
)

_xa = pl.pallas_call(
    _xa_body,
    out_shape=jax.ShapeDtypeStruct((N, H), jnp.float32),
)

_div = pl.pallas_call(
    _div_body,
    out_shape=jax.ShapeDtypeStruct((N * DI // 128, 128), jnp.float32),
)


def _block_diag8(w):
    k, m = w.shape
    out = jnp.zeros((8 * k, 8 * m), w.dtype)
    for j in range(8):
        out = lax.dynamic_update_slice(out, w, (j * k, j * m))
    return out


def kernel(x, edge_index, edge_attr, W1, b1, W2, b2):
    row = edge_index[0]
    col = edge_index[1]
    W1big = _block_diag8(W1[D:])
    W2big = _block_diag8(W2)
    b2w = jnp.tile(b2, 8).reshape(1, 8 * DI)

    xa = _xa(x, W1[:D], b1.reshape(1, H))
    g = _gather(xa, row)
    g_wide = g.reshape(EW, 8 * H)
    ea_wide = edge_attr.reshape(EW, 8 * DE)
    en_wide = _mlp(g_wide, ea_wide, W1big, W2big, b2w)
    edge_new = en_wide.reshape(E, DI)
    col3 = col.reshape(E // CH, NSTR, STR)
    sums, cnts = _scatter(col3, edge_new)
    node_wide = _div(sums.reshape(NC, N * DI // 128, 128),
                     cnts.reshape(NC, N * DI // 128, 128))
    node_new = node_wide.reshape(N, DI)
    return (node_new, edge_new)


# speedup vs baseline: 5.6513x; 1.6020x over previous
import jax
import jax.numpy as jnp
from jax import lax
from jax.experimental import pallas as pl
from jax.experimental.pallas import tpu as pltpu
from jax.experimental.pallas import tpu_sc as plsc

NC, NS = 2, 16
NW = NC * NS
STR = 80
NSTR = 5
CH = STR * NSTR
NCHUNK = 25

N = 10000
E = 320000
D = 128
DE = 16
H = 128
DI = 16

EPW = E // NW
ZR = 1000
NZW = N // ZR

BE = 3200


def _xa_body(x_ref, w_ref, b_ref, o_ref):
    o_ref[...] = (
        jnp.dot(x_ref[...], w_ref[...], preferred_element_type=jnp.float32)
        + b_ref[...]
    )


def _mlp_body(g_ref, eat_ref, w1b_ref, w2p_ref, b2p_ref, o_ref):
    eb = lax.dot_general(
        eat_ref[...], w1b_ref[...], (((0,), (0,)), ((), ())),
        preferred_element_type=jnp.float32,
    )
    h = jnp.maximum(g_ref[...] + eb, 0.0)
    o_ref[...] = (
        jnp.dot(h, w2p_ref[...], preferred_element_type=jnp.float32) + b2p_ref[...]
    )


def _div_body(s_ref, c_ref, o_ref):
    sm = s_ref[0] + s_ref[1]
    ct = c_ref[0] + c_ref[1]
    o_ref[...] = sm / jnp.maximum(ct, 1.0)


def _gather_body(xa_hbm, idx_hbm, g_hbm, idx_v, rows_v, gsem, wsem):
    c = lax.axis_index("c")
    s = lax.axis_index("s")
    base = (c * NS + s) * EPW

    wb = [None, None]
    for t in range(NCHUNK):
        b = t % 2
        if t >= 2:
            wb[b].wait()
        e0 = base + t * CH
        pltpu.sync_copy(idx_hbm.at[pl.ds(e0, CH)], idx_v.at[b])
        fired = []
        for k in range(NSTR):
            fired.append(pltpu.async_copy(
                xa_hbm.at[idx_v.at[b].at[pl.ds(k * STR, STR)]],
                rows_v.at[b].at[pl.ds(k * STR, STR)],
                gsem.at[b],
            ))
        for f in fired:
            f.wait()
        wb[b] = pltpu.async_copy(rows_v.at[b], g_hbm.at[pl.ds(e0, CH)], wsem.at[b])
    wb[(NCHUNK - 2) % 2].wait()
    wb[(NCHUNK - 1) % 2].wait()


def _repack_out(acc, t1, t2, out_hbm, core, part):
    ext = 290 if part == 3 else 320
    for half in range(2):
        hw = min(ext - 160 * half, 160)
        w0 = part * 320 + 160 * half
        pltpu.sync_copy(acc.at[pl.ds(w0 * 8, hw * 8)], t1.at[pl.ds(0, hw * 8)])

        def rows(q, carry):
            for j in range(8):
                t2[q, pl.ds(j * 16, 16)] = t1[q * 8 + j]
            return carry

        lax.fori_loop(0, hw, rows, 0)
        pltpu.sync_copy(t2.at[pl.ds(0, hw)], out_hbm.at[core, pl.ds(w0, hw)])


def _scatter_body(col_hbm, en_hbm, sums_hbm, cnts_hbm,
                  cidx_v, dat_v, one_v, zer_v, t1_v, t2_v, acc_s, acc_c, ssem, asem):
    c = lax.axis_index("c")
    s = lax.axis_index("s")
    base = (c * NS + s) * EPW

    def fill_z(i, carry):
        zer_v[i] = jnp.zeros((16,), jnp.float32)
        return carry

    lax.fori_loop(0, ZR, fill_z, 0)

    def fill_o(i, carry):
        one_v[i] = jnp.full((16,), 1.0, jnp.float32)
        return carry

    lax.fori_loop(0, STR, fill_o, 0)

    @pl.when(s < NZW)
    def _():
        pltpu.sync_copy(zer_v, acc_s.at[pl.ds(s * ZR, ZR)])
        pltpu.sync_copy(zer_v, acc_c.at[pl.ds(s * ZR, ZR)])

    plsc.subcore_barrier()

    adds = [None, None]
    for t in range(NCHUNK):
        b = t % 2
        if t >= 2:
            for f in adds[b]:
                f.wait()
        e0 = base + t * CH
        st1 = pltpu.async_copy(col_hbm.at[pl.ds(e0, CH)], cidx_v.at[b], ssem.at[b])
        st2 = pltpu.async_copy(en_hbm.at[pl.ds(e0, CH), pl.ds(0, DI)], dat_v.at[b],
                               ssem.at[b])
        st1.wait()
        st2.wait()
        fired = []
        for k in range(NSTR):
            ck = cidx_v.at[b].at[pl.ds(k * STR, STR)]
            fired.append(pltpu.async_copy(
                dat_v.at[b].at[pl.ds(k * STR, STR)],
                acc_s.at[ck], asem.at[b], add=True))
            fired.append(pltpu.async_copy(
                one_v, acc_c.at[ck], asem.at[b], add=True))
        adds[b] = fired
    for b in ((NCHUNK - 2) % 2, (NCHUNK - 1) % 2):
        for f in adds[b]:
            f.wait()
    plsc.subcore_barrier()

    for part in range(4):
        @pl.when(s == part)
        def _(part=part):
            _repack_out(acc_s, t1_v, t2_v, sums_hbm, c, part)

        @pl.when(s == 4 + part)
        def _(part=part):
            _repack_out(acc_c, t1_v, t2_v, cnts_hbm, c, part)


_sc_mesh = plsc.VectorSubcoreMesh(
    core_axis_name="c", subcore_axis_name="s", num_cores=NC, num_subcores=NS
)
_sc_params = pltpu.CompilerParams(use_tc_tiling_on_sc=False)

_gather = pl.kernel(
    _gather_body,
    compiler_params=_sc_params,
    out_type=jax.ShapeDtypeStruct((E, H), jnp.float32),
    mesh=_sc_mesh,
    scratch_types=[
        pltpu.VMEM((2, CH), jnp.int32),
        pltpu.VMEM((2, CH, H), jnp.float32),
        pltpu.SemaphoreType.DMA((2,)),
        pltpu.SemaphoreType.DMA((2,)),
    ],
)

_scatter = pl.kernel(
    _scatter_body,
    compiler_params=_sc_params,
    out_type=(
        jax.ShapeDtypeStruct((NC, N * DI // 128, 128), jnp.float32),
        jax.ShapeDtypeStruct((NC, N * DI // 128, 128), jnp.float32),
    ),
    mesh=_sc_mesh,
    scratch_types=[
        pltpu.VMEM((2, CH), jnp.int32),
        pltpu.VMEM((2, CH, DI), jnp.float32),
        pltpu.VMEM((STR, DI), jnp.float32),
        pltpu.VMEM((ZR, DI), jnp.float32),
        pltpu.VMEM((1280, DI), jnp.float32),
        pltpu.VMEM((160, 128), jnp.float32),
        pltpu.VMEM_SHARED((N, DI), jnp.float32),
        pltpu.VMEM_SHARED((N, DI), jnp.float32),
        pltpu.SemaphoreType.DMA((2,)),
        pltpu.SemaphoreType.DMA((2,)),
    ],
)

_mlp = pl.pallas_call(
    _mlp_body,
    grid=(E // BE,),
    in_specs=[
        pl.BlockSpec((BE, H), lambda i: (i, 0)),
        pl.BlockSpec((DE, BE), lambda i: (0, i)),
        pl.BlockSpec((DE, H), lambda i: (0, 0)),
        pl.BlockSpec((H, H), lambda i: (0, 0)),
        pl.BlockSpec((1, H), lambda i: (0, 0)),
    ],
    out_specs=pl.BlockSpec((BE, H), lambda i: (i, 0)),
    out_shape=jax.ShapeDtypeStruct((E, H), jnp.float32),
)

_xa = pl.pallas_call(
    _xa_body,
    out_shape=jax.ShapeDtypeStruct((N, H), jnp.float32),
)

_div = pl.pallas_call(
    _div_body,
    out_shape=jax.ShapeDtypeStruct((N * DI // 128, 128), jnp.float32),
)


def kernel(x, edge_index, edge_attr, W1, b1, W2, b2):
    row = edge_index[0]
    col = edge_index[1]

    xa = _xa(x, W1[:D], b1.reshape(1, H))
    g = _gather(xa, row)
    eat = edge_attr.T
    w2p = jnp.concatenate([W2, jnp.zeros((H, H - DI), jnp.float32)], axis=1)
    b2p = jnp.concatenate([b2, jnp.zeros((H - DI,), jnp.float32)]).reshape(1, H)
    en_pad = _mlp(g, eat, W1[D:], w2p, b2p)
    edge_new = en_pad[:, :DI]
    sums, cnts = _scatter(col, en_pad)
    node_wide = _div(sums, cnts)
    node_new = node_wide.reshape(N, DI)
    return (node_new, edge_new)
